# baseline (speedup 1.0000x reference)
import functools

import jax
import jax.numpy as jnp
from jax import lax
from jax.experimental import pallas as pl
from jax.experimental.pallas import tpu as pltpu
from jax.experimental.pallas import tpu_sc as plsc

N_USERS = 100000
N_ITEMS = 100000
N_GENRES = 32
K = 64
BATCH = 4096

NC, NS, L = 2, 16, 16
NW = NC * NS

CW = 1280
NCHUNK = N_USERS // CW
TAIL0 = NCHUNK * CW
TAIL_W = N_USERS - TAIL0
ROUNDS = -(-NCHUNK // NW)
LIST_N = BATCH + L


def _scan_ids(ids_v, c0, width, lst_v):
    lanes = lax.iota(jnp.int32, L)

    def scan(j, off):
        v = ids_v[pl.ds(j * L, L)]
        rel = v - c0
        m = (rel >= 0) & (rel < width)
        cnt = plsc.all_reduce_population_count(m)[0]
        packed = rel | ((j * L + lanes) << 16)
        plsc.store_compressed(lst_v.at[pl.ds(off, L)], packed, mask=m)
        return off + cnt

    return lax.fori_loop(0, BATCH // L, scan, 0)


def _scan_ids3(ids_v, base, lsts):
    lanes = lax.iota(jnp.int32, L)

    def scan(j, offs):
        v = ids_v[pl.ds(j * L, L)]
        bidx = (j * L + lanes) << 16
        out = []
        for r in range(ROUNDS):
            c0 = (base + r * NW) * CW
            rel = v - c0
            m = (rel >= 0) & (rel < CW)
            cnt = plsc.all_reduce_population_count(m)[0]
            plsc.store_compressed(lsts[r].at[pl.ds(offs[r], L)], rel | bidx,
                                  mask=m)
            out.append(offs[r] + cnt)
        return tuple(out)

    return lax.fori_loop(0, BATCH // L, scan, (0,) * ROUNDS)


def _extract_matches(out_hbm, buf_v, lst_v, stage_v, ssem, nmatch):
    lanes = lax.iota(jnp.int32, L)
    ngroups = (nmatch + L - 1) // L
    p0 = lst_v[pl.ds(0, L)][0]
    lst_v[pl.ds(nmatch, L)] = jnp.zeros((L,), jnp.int32) + p0

    def extract(g, _):
        packed = lst_v[pl.ds(g * L, L)]
        rel = packed & 0xFFFF
        rows = lax.shift_right_logical(packed, 16)
        for k in range(K):
            kv = jnp.zeros((L,), jnp.int32) + k
            vals = plsc.load_gather(buf_v, [kv, rel])
            plsc.store_scatter(stage_v, [lanes, kv], vals)
        pltpu.async_copy(stage_v, out_hbm.at[rows], ssem).wait()
        return _

    lax.fori_loop(0, ngroups, extract, 0)


def _extract_table(tab_hbm, tail_hbm, ids_v, out_hbm, chunk_v, tail_v,
                   lsts, stage_v, csem, ssem, wid, shift, tail_wid):
    base = lax.rem(wid + shift, NW)
    c0_first = pl.multiple_of(base * CW, CW)
    cp_first = pltpu.async_copy(tab_hbm.at[:, pl.ds(c0_first, CW)],
                                chunk_v, csem)
    offs = _scan_ids3(ids_v, base, lsts)
    for r in range(ROUNDS):
        c = base + r * NW

        @pl.when(c < NCHUNK)
        def _go(r=r, c=c):
            if r > 0:
                c0 = pl.multiple_of(c * CW, CW)
                pltpu.async_copy(tab_hbm.at[:, pl.ds(c0, CW)],
                                 chunk_v, csem).wait()
            else:
                cp_first.wait()
            _extract_matches(out_hbm, chunk_v, lsts[r], stage_v, ssem,
                             offs[r])

    @pl.when(wid == tail_wid)
    def _tail():
        cp = pltpu.async_copy(tail_hbm, tail_v, csem)
        n = _scan_ids(ids_v, TAIL0, TAIL_W, lsts[0])
        cp.wait()
        _extract_matches(out_hbm, tail_v, lsts[0], stage_v, ssem, n)


def _sc_extract_body(p_hbm, q_hbm, pt_hbm, qt_hbm, u_hbm, i_hbm,
                     pu_hbm, qi_hbm,
                     u_v, i_v, chunk_v, tail_v, lst0_v, lst1_v, lst2_v,
                     stage_v, csem, ssem):
    wid = lax.axis_index("s") * NC + lax.axis_index("c")
    pltpu.sync_copy(u_hbm, u_v)
    pltpu.sync_copy(i_hbm, i_v)
    lsts = (lst0_v, lst1_v, lst2_v)
    _extract_table(p_hbm, pt_hbm, u_v, pu_hbm, chunk_v, tail_v,
                   lsts, stage_v, csem, ssem, wid, 0, NW - 1)
    _extract_table(q_hbm, qt_hbm, i_v, qi_hbm, chunk_v, tail_v,
                   lsts, stage_v, csem, ssem, wid, NW // 2, NW - 2)


@functools.cache
def _sc_extract():
  return pl.kernel(
    _sc_extract_body,
    out_type=(jax.ShapeDtypeStruct((BATCH, 2 * K), jnp.float32),
              jax.ShapeDtypeStruct((BATCH, 2 * K), jnp.float32)),
    mesh=plsc.VectorSubcoreMesh(core_axis_name="c", subcore_axis_name="s",
                                num_cores=NC, num_subcores=NS),
    scratch_types=[
        pltpu.VMEM((BATCH,), jnp.int32),
        pltpu.VMEM((BATCH,), jnp.int32),
        pltpu.VMEM((K, CW), jnp.float32),
        pltpu.VMEM((K, TAIL_W), jnp.float32),
        pltpu.VMEM((LIST_N,), jnp.int32),
        pltpu.VMEM((LIST_N,), jnp.int32),
        pltpu.VMEM((LIST_N,), jnp.int32),
        pltpu.VMEM((L, 2 * K), jnp.float32),
        pltpu.SemaphoreType.DMA,
        pltpu.SemaphoreType.DMA,
    ],
    compiler_params=pltpu.CompilerParams(needs_layout_passes=False),
  )


BLK_B = 512


def _dot(a, b, dims):
    return lax.dot_general(a, b, (dims, ((), ())),
                           preferred_element_type=jnp.float32)


def _mlp_body(pu_ref, qi_ref, g_ref, w1_ref, b1_ref, w2_ref, b2_ref,
              w3_ref, b3_ref, hw_ref, qg_ref, o_ref):
    pu = pu_ref[...][:, :K]
    qi_item = qi_ref[...][:, :K]
    qi = qi_item + _dot(g_ref[...], qg_ref[...], ((1,), (1,)))
    x = jnp.concatenate([pu, qi], axis=1)
    z1 = _dot(x, w1_ref[...], ((1,), (1,))) + b1_ref[...]
    x1 = jnp.maximum(z1, 0.0)
    x2 = jnp.maximum(_dot(x1, w2_ref[...], ((1,), (1,))) + b2_ref[...], 0.0)
    x3 = jnp.maximum(_dot(x2, w3_ref[...], ((1,), (1,))) + b3_ref[...], 0.0)
    o_ref[...] = _dot(x3, hw_ref[...], ((1,), (1,)))


def _mlp_call(urows, irows, genres, W1, b1, W2, b2, W3, b3, h_w, qg):
    full = lambda shape: pl.BlockSpec(shape, lambda i: (0,) * len(shape))
    return pl.pallas_call(
        _mlp_body,
        grid=(BATCH // BLK_B,),
        in_specs=[
            pl.BlockSpec((BLK_B, 2 * K), lambda i: (i, 0)),
            pl.BlockSpec((BLK_B, 2 * K), lambda i: (i, 0)),
            pl.BlockSpec((BLK_B, N_GENRES), lambda i: (i, 0)),
            full(W1.shape), full((1, 256)),
            full(W2.shape), full((1, 128)),
            full(W3.shape), full((1, 64)),
            full(h_w.shape), full(qg.shape),
        ],
        out_specs=pl.BlockSpec((BLK_B, 1), lambda i: (i, 0)),
        out_shape=jax.ShapeDtypeStruct((BATCH, 1), jnp.float32),
    )(urows, irows, genres, W1, b1.reshape(1, -1), W2, b2.reshape(1, -1),
      W3, b3.reshape(1, -1), h_w, qg)


def kernel(user_ids, item_ids, genres_one_hot, P_w, Q_w, W1, b1, W2, b2, W3, b3, h_w):
    qg = Q_w[:, N_ITEMS:]
    urows, irows = _sc_extract()(P_w, Q_w,
                                 P_w[:, TAIL0:],
                                 Q_w[:, TAIL0:N_ITEMS],
                                 user_ids.astype(jnp.int32),
                                 item_ids.astype(jnp.int32))
    return _mlp_call(urows, irows, genres_one_hot, W1, b1, W2, b2, W3, b3,
                     h_w, qg)


# speedup vs baseline: 1.0772x; 1.0772x over previous
import functools

import jax
import jax.numpy as jnp
from jax import lax
from jax.experimental import pallas as pl
from jax.experimental.pallas import tpu as pltpu
from jax.experimental.pallas import tpu_sc as plsc

N_USERS = 100000
N_ITEMS = 100000
N_GENRES = 32
K = 64
BATCH = 4096

NC, NS, L = 2, 16, 16
NW = NC * NS

CW = 640
NCHUNK = N_USERS // CW
TAIL0 = NCHUNK * CW
TAIL_W = N_USERS - TAIL0
ROUNDS = -(-NCHUNK // NW)
LIST_N = BATCH + L


def _scan_ids(ids_v, c0, width, lst_v):
    lanes = lax.iota(jnp.int32, L)

    def scan(j, off):
        v = ids_v[pl.ds(j * L, L)]
        rel = v - c0
        m = (rel >= 0) & (rel < width)
        cnt = plsc.all_reduce_population_count(m)[0]
        packed = rel | ((j * L + lanes) << 16)
        plsc.store_compressed(lst_v.at[pl.ds(off, L)], packed, mask=m)
        return off + cnt

    return lax.fori_loop(0, BATCH // L, scan, 0)


def _scan_ids3(ids_v, base, lsts):
    lanes = lax.iota(jnp.int32, L)

    def scan(j, offs):
        v = ids_v[pl.ds(j * L, L)]
        bidx = (j * L + lanes) << 16
        out = []
        for r in range(ROUNDS):
            c0 = (base + r * NW) * CW
            rel = v - c0
            m = (rel >= 0) & (rel < CW)
            cnt = plsc.all_reduce_population_count(m)[0]
            plsc.store_compressed(lsts[r].at[pl.ds(offs[r], L)], rel | bidx,
                                  mask=m)
            out.append(offs[r] + cnt)
        return tuple(out)

    return lax.fori_loop(0, BATCH // L, scan, (0,) * ROUNDS)


def _extract_matches(out_hbm, buf_v, lst_v, stage_v, ssem, nmatch):
    lanes = lax.iota(jnp.int32, L)
    ngroups = (nmatch + L - 1) // L
    p0 = lst_v[pl.ds(0, L)][0]
    lst_v[pl.ds(nmatch, L)] = jnp.zeros((L,), jnp.int32) + p0

    def extract(g, _):
        packed = lst_v[pl.ds(g * L, L)]
        rel = packed & 0xFFFF
        rows = lax.shift_right_logical(packed, 16)
        for k in range(K):
            kv = jnp.zeros((L,), jnp.int32) + k
            vals = plsc.load_gather(buf_v, [kv, rel])
            plsc.store_scatter(stage_v, [lanes, kv], vals)
        pltpu.async_copy(stage_v, out_hbm.at[rows], ssem).wait()
        return _

    lax.fori_loop(0, ngroups, extract, 0)


def _extract_table(tab_hbm, tail_hbm, ids_v, out_hbm, chunk_bufs, tail_v,
                   lsts, stage_v, csem, ssem, wid, shift, tail_wid):
    base = lax.rem(wid + shift, NW)
    bufs = chunk_bufs
    sems = csem

    def _chunk_copy(r):
        c0 = pl.multiple_of((base + r * NW) * CW, CW)
        return pltpu.make_async_copy(tab_hbm.at[:, pl.ds(c0, CW)],
                                     bufs[r % 2], sems[r % 2])

    _chunk_copy(0).start()
    offs = _scan_ids3(ids_v, base, lsts)
    for r in range(ROUNDS):
        c = base + r * NW

        @pl.when(c < NCHUNK)
        def _go(r=r, c=c):
            if r + 1 < ROUNDS:
                @pl.when(c + NW < NCHUNK)
                def _pf():
                    _chunk_copy(r + 1).start()

            _chunk_copy(r).wait()
            _extract_matches(out_hbm, bufs[r % 2], lsts[r], stage_v, ssem,
                             offs[r])

    @pl.when(wid == tail_wid)
    def _tail():
        cp = pltpu.async_copy(tail_hbm, tail_v, sems[0])
        n = _scan_ids(ids_v, TAIL0, TAIL_W, lsts[0])
        cp.wait()
        _extract_matches(out_hbm, tail_v, lsts[0], stage_v, ssem, n)


def _sc_extract_body(p_hbm, q_hbm, pt_hbm, qt_hbm, u_hbm, i_hbm,
                     pu_hbm, qi_hbm,
                     u_v, i_v, chunk0_v, chunk1_v, tail_v,
                     lst0_v, lst1_v, lst2_v, lst3_v, lst4_v,
                     stage_v, csem0, csem1, ssem):
    wid = lax.axis_index("s") * NC + lax.axis_index("c")
    pltpu.sync_copy(u_hbm, u_v)
    pltpu.sync_copy(i_hbm, i_v)
    lsts = (lst0_v, lst1_v, lst2_v, lst3_v, lst4_v)
    bufs = (chunk0_v, chunk1_v)
    sems = (csem0, csem1)
    _extract_table(p_hbm, pt_hbm, u_v, pu_hbm, bufs, tail_v,
                   lsts, stage_v, sems, ssem, wid, 0, NW - 1)
    _extract_table(q_hbm, qt_hbm, i_v, qi_hbm, bufs, tail_v,
                   lsts, stage_v, sems, ssem, wid, NW // 2, NW - 2)


@functools.cache
def _sc_extract():
  return pl.kernel(
    _sc_extract_body,
    out_type=(jax.ShapeDtypeStruct((BATCH, 2 * K), jnp.float32),
              jax.ShapeDtypeStruct((BATCH, 2 * K), jnp.float32)),
    mesh=plsc.VectorSubcoreMesh(core_axis_name="c", subcore_axis_name="s",
                                num_cores=NC, num_subcores=NS),
    scratch_types=[
        pltpu.VMEM((BATCH,), jnp.int32),
        pltpu.VMEM((BATCH,), jnp.int32),
        pltpu.VMEM((K, CW), jnp.float32),
        pltpu.VMEM((K, CW), jnp.float32),
        pltpu.VMEM((K, TAIL_W), jnp.float32),
        pltpu.VMEM((LIST_N,), jnp.int32),
        pltpu.VMEM((LIST_N,), jnp.int32),
        pltpu.VMEM((LIST_N,), jnp.int32),
        pltpu.VMEM((LIST_N,), jnp.int32),
        pltpu.VMEM((LIST_N,), jnp.int32),
        pltpu.VMEM((L, 2 * K), jnp.float32),
        pltpu.SemaphoreType.DMA,
        pltpu.SemaphoreType.DMA,
        pltpu.SemaphoreType.DMA,
    ],
    compiler_params=pltpu.CompilerParams(needs_layout_passes=False),
  )


BLK_B = 512


def _dot(a, b, dims):
    return lax.dot_general(a, b, (dims, ((), ())),
                           preferred_element_type=jnp.float32)


def _mlp_body(pu_ref, qi_ref, g_ref, w1_ref, b1_ref, w2_ref, b2_ref,
              w3_ref, b3_ref, hw_ref, qg_ref, o_ref):
    pu = pu_ref[...][:, :K]
    qi_item = qi_ref[...][:, :K]
    qi = qi_item + _dot(g_ref[...], qg_ref[...], ((1,), (1,)))
    x = jnp.concatenate([pu, qi], axis=1)
    z1 = _dot(x, w1_ref[...], ((1,), (1,))) + b1_ref[...]
    x1 = jnp.maximum(z1, 0.0)
    x2 = jnp.maximum(_dot(x1, w2_ref[...], ((1,), (1,))) + b2_ref[...], 0.0)
    x3 = jnp.maximum(_dot(x2, w3_ref[...], ((1,), (1,))) + b3_ref[...], 0.0)
    o_ref[...] = _dot(x3, hw_ref[...], ((1,), (1,)))


def _mlp_call(urows, irows, genres, W1, b1, W2, b2, W3, b3, h_w, qg):
    full = lambda shape: pl.BlockSpec(shape, lambda i: (0,) * len(shape))
    return pl.pallas_call(
        _mlp_body,
        grid=(BATCH // BLK_B,),
        in_specs=[
            pl.BlockSpec((BLK_B, 2 * K), lambda i: (i, 0)),
            pl.BlockSpec((BLK_B, 2 * K), lambda i: (i, 0)),
            pl.BlockSpec((BLK_B, N_GENRES), lambda i: (i, 0)),
            full(W1.shape), full((1, 256)),
            full(W2.shape), full((1, 128)),
            full(W3.shape), full((1, 64)),
            full(h_w.shape), full(qg.shape),
        ],
        out_specs=pl.BlockSpec((BLK_B, 1), lambda i: (i, 0)),
        out_shape=jax.ShapeDtypeStruct((BATCH, 1), jnp.float32),
    )(urows, irows, genres, W1, b1.reshape(1, -1), W2, b2.reshape(1, -1),
      W3, b3.reshape(1, -1), h_w, qg)


def kernel(user_ids, item_ids, genres_one_hot, P_w, Q_w, W1, b1, W2, b2, W3, b3, h_w):
    qg = Q_w[:, N_ITEMS:]
    urows, irows = _sc_extract()(P_w, Q_w,
                                 P_w[:, TAIL0:],
                                 Q_w[:, TAIL0:N_ITEMS],
                                 user_ids.astype(jnp.int32),
                                 item_ids.astype(jnp.int32))
    return _mlp_call(urows, irows, genres_one_hot, W1, b1, W2, b2, W3, b3,
                     h_w, qg)


# speedup vs baseline: 1.1225x; 1.0420x over previous
import functools

import jax
import jax.numpy as jnp
from jax import lax
from jax.experimental import pallas as pl
from jax.experimental.pallas import tpu as pltpu
from jax.experimental.pallas import tpu_sc as plsc

N_USERS = 100000
N_ITEMS = 100000
N_GENRES = 32
K = 64
BATCH = 4096

NC, NS, L = 2, 16, 16
NW = NC * NS

CW = 640
NCHUNK = N_USERS // CW
TAIL0 = NCHUNK * CW
TAIL_W = N_USERS - TAIL0
ROUNDS = -(-NCHUNK // NW)
LIST_N = BATCH + L


def _scan_ids(ids_v, c0, width, lst_v):
    lanes = lax.iota(jnp.int32, L)

    def scan(j, off):
        v = ids_v[pl.ds(j * L, L)]
        rel = v - c0
        m = (rel >= 0) & (rel < width)
        cnt = plsc.all_reduce_population_count(m)[0]
        packed = rel | ((j * L + lanes) << 16)
        plsc.store_compressed(lst_v.at[pl.ds(off, L)], packed, mask=m)
        return off + cnt

    return lax.fori_loop(0, BATCH // L, scan, 0)


def _scan_ids3(ids_v, base, lsts):
    lanes = lax.iota(jnp.int32, L)

    def scan(j, offs):
        v = ids_v[pl.ds(j * L, L)]
        bidx = (j * L + lanes) << 16
        out = []
        for r in range(ROUNDS):
            c0 = (base + r * NW) * CW
            rel = v - c0
            m = (rel >= 0) & (rel < CW)
            cnt = plsc.all_reduce_population_count(m)[0]
            plsc.store_compressed(lsts[r].at[pl.ds(offs[r], L)], rel | bidx,
                                  mask=m)
            out.append(offs[r] + cnt)
        return tuple(out)

    return lax.fori_loop(0, BATCH // L, scan, (0,) * ROUNDS)


def _extract_matches(out_hbm, buf_v, lst_v, stage_v, ssem, nmatch):
    lanes = lax.iota(jnp.int32, L)
    ngroups = (nmatch + L - 1) // L
    p0 = lst_v[pl.ds(0, L)][0]
    lst_v[pl.ds(nmatch, L)] = jnp.zeros((L,), jnp.int32) + p0

    def extract(g, _):
        packed = lst_v[pl.ds(g * L, L)]
        rel = packed & 0xFFFF
        rows = lax.shift_right_logical(packed, 16)
        for k in range(K):
            kv = jnp.zeros((L,), jnp.int32) + k
            vals = plsc.load_gather(buf_v, [kv, rel])
            plsc.store_scatter(stage_v, [lanes, kv], vals)
        pltpu.async_copy(stage_v, out_hbm.at[rows], ssem).wait()
        return _

    lax.fori_loop(0, ngroups, extract, 0)


def _extract_table(tab_hbm, tail_hbm, ids_v, out_hbm, chunk_bufs, tail_v,
                   lsts, stage_v, csem, ssem, wid, shift, tail_wid):
    base = lax.rem(wid + shift, NW)
    bufs = chunk_bufs
    sems = csem

    def _chunk_copy(r):
        c0 = pl.multiple_of((base + r * NW) * CW, CW)
        return pltpu.make_async_copy(tab_hbm.at[:, pl.ds(c0, CW)],
                                     bufs[r % 2], sems[r % 2])

    _chunk_copy(0).start()
    offs = _scan_ids3(ids_v, base, lsts)
    for r in range(ROUNDS):
        c = base + r * NW

        @pl.when(c < NCHUNK)
        def _go(r=r, c=c):
            if r + 1 < ROUNDS:
                @pl.when(c + NW < NCHUNK)
                def _pf():
                    _chunk_copy(r + 1).start()

            _chunk_copy(r).wait()
            _extract_matches(out_hbm, bufs[r % 2], lsts[r], stage_v, ssem,
                             offs[r])

    @pl.when(wid == tail_wid)
    def _tail():
        cp = pltpu.async_copy(tail_hbm, tail_v, sems[0])
        n = _scan_ids(ids_v, TAIL0, TAIL_W, lsts[0])
        cp.wait()
        _extract_matches(out_hbm, tail_v, lsts[0], stage_v, ssem, n)


def _sc_extract_body(p_hbm, q_hbm, pt_hbm, qt_hbm, u_hbm, i_hbm,
                     pu_hbm, qi_hbm,
                     u_v, i_v, chunk0_v, chunk1_v, tail_v,
                     lst0_v, lst1_v, lst2_v, lst3_v, lst4_v,
                     stage_v, csem0, csem1, ssem):
    wid = lax.axis_index("s") * NC + lax.axis_index("c")
    pltpu.sync_copy(u_hbm, u_v)
    pltpu.sync_copy(i_hbm, i_v)
    lsts = (lst0_v, lst1_v, lst2_v, lst3_v, lst4_v)
    bufs = (chunk0_v, chunk1_v)
    sems = (csem0, csem1)
    _extract_table(p_hbm, pt_hbm, u_v, pu_hbm, bufs, tail_v,
                   lsts, stage_v, sems, ssem, wid, 0, NW - 1)
    _extract_table(q_hbm, qt_hbm, i_v, qi_hbm, bufs, tail_v,
                   lsts, stage_v, sems, ssem, wid, NW // 2, NW - 2)


@functools.cache
def _sc_extract():
  return pl.kernel(
    _sc_extract_body,
    out_type=(jax.ShapeDtypeStruct((BATCH, 2 * K), jnp.float32),
              jax.ShapeDtypeStruct((BATCH, 2 * K), jnp.float32)),
    mesh=plsc.VectorSubcoreMesh(core_axis_name="c", subcore_axis_name="s",
                                num_cores=NC, num_subcores=NS),
    scratch_types=[
        pltpu.VMEM((BATCH,), jnp.int32),
        pltpu.VMEM((BATCH,), jnp.int32),
        pltpu.VMEM((K, CW), jnp.float32),
        pltpu.VMEM((K, CW), jnp.float32),
        pltpu.VMEM((K, TAIL_W), jnp.float32),
        pltpu.VMEM((LIST_N,), jnp.int32),
        pltpu.VMEM((LIST_N,), jnp.int32),
        pltpu.VMEM((LIST_N,), jnp.int32),
        pltpu.VMEM((LIST_N,), jnp.int32),
        pltpu.VMEM((LIST_N,), jnp.int32),
        pltpu.VMEM((L, 2 * K), jnp.float32),
        pltpu.SemaphoreType.DMA,
        pltpu.SemaphoreType.DMA,
        pltpu.SemaphoreType.DMA,
    ],
    compiler_params=pltpu.CompilerParams(needs_layout_passes=False),
  )


BLK_B = 512


def _dot(a, b, dims):
    return lax.dot_general(a, b, (dims, ((), ())),
                           preferred_element_type=jnp.float32)


def _mlp_body(pu_ref, qi_ref, g_ref, w1_ref, b1_ref, w2_ref, b2_ref,
              w3_ref, b3_ref, hw_ref, qg_ref, o_ref):
    pu_t = jnp.transpose(pu_ref[...][:, :K])
    qi_item_t = jnp.transpose(qi_ref[...][:, :K])
    qi_t = qi_item_t + _dot(qg_ref[...], g_ref[...], ((1,), (1,)))
    x_t = jnp.concatenate([pu_t, qi_t], axis=0)
    z1 = _dot(w1_ref[...], x_t, ((1,), (0,))) + b1_ref[...]
    x1 = jnp.maximum(z1, 0.0)
    x2 = jnp.maximum(_dot(w2_ref[...], x1, ((1,), (0,))) + b2_ref[...], 0.0)
    x3 = jnp.maximum(_dot(w3_ref[...], x2, ((1,), (0,))) + b3_ref[...], 0.0)
    o_ref[...] = _dot(hw_ref[...], x3, ((1,), (0,)))


def _mlp_call(urows, irows, genres, W1, b1, W2, b2, W3, b3, h_w, qg):
    full = lambda shape: pl.BlockSpec(shape, lambda i: (0,) * len(shape))
    return pl.pallas_call(
        _mlp_body,
        grid=(BATCH // BLK_B,),
        in_specs=[
            pl.BlockSpec((BLK_B, 2 * K), lambda i: (i, 0)),
            pl.BlockSpec((BLK_B, 2 * K), lambda i: (i, 0)),
            pl.BlockSpec((BLK_B, N_GENRES), lambda i: (i, 0)),
            full(W1.shape), full((256, 1)),
            full(W2.shape), full((128, 1)),
            full(W3.shape), full((64, 1)),
            full(h_w.shape), full(qg.shape),
        ],
        out_specs=pl.BlockSpec((1, BLK_B), lambda i: (0, i)),
        out_shape=jax.ShapeDtypeStruct((1, BATCH), jnp.float32),
    )(urows, irows, genres, W1, b1.reshape(-1, 1), W2, b2.reshape(-1, 1),
      W3, b3.reshape(-1, 1), h_w, qg)


def kernel(user_ids, item_ids, genres_one_hot, P_w, Q_w, W1, b1, W2, b2, W3, b3, h_w):
    qg = Q_w[:, N_ITEMS:]
    urows, irows = _sc_extract()(P_w, Q_w,
                                 P_w[:, TAIL0:],
                                 Q_w[:, TAIL0:N_ITEMS],
                                 user_ids.astype(jnp.int32),
                                 item_ids.astype(jnp.int32))
    out = _mlp_call(urows, irows, genres_one_hot, W1, b1, W2, b2, W3, b3,
                    h_w, qg)
    return out.reshape(BATCH, 1)


# speedup vs baseline: 1.1815x; 1.0526x over previous
import functools

import jax
import jax.numpy as jnp
from jax import lax
from jax.experimental import pallas as pl
from jax.experimental.pallas import tpu as pltpu
from jax.experimental.pallas import tpu_sc as plsc

N_USERS = 100000
N_ITEMS = 100000
N_GENRES = 32
K = 64
BATCH = 4096

NC, NS, L = 2, 16, 16
NW = NC * NS

CW = 640
NCHUNK = N_USERS // CW
TAIL0 = NCHUNK * CW
TAIL_W = N_USERS - TAIL0
ROUNDS = -(-NCHUNK // NW)
LIST_N = BATCH + L


def _scan_ids(ids_v, c0, width, lst_v):
    lanes = lax.iota(jnp.int32, L)

    def scan(j, off):
        v = ids_v[pl.ds(j * L, L)]
        rel = v - c0
        m = (rel >= 0) & (rel < width)
        cnt = plsc.all_reduce_population_count(m)[0]
        packed = rel | ((j * L + lanes) << 16)
        plsc.store_compressed(lst_v.at[pl.ds(off, L)], packed, mask=m)
        return off + cnt

    return lax.fori_loop(0, BATCH // L, scan, 0)


def _scan_ids3(ids_v, base, lsts):
    lanes = lax.iota(jnp.int32, L)

    def scan(j, offs):
        v = ids_v[pl.ds(j * L, L)]
        bidx = (j * L + lanes) << 16
        out = []
        for r in range(ROUNDS):
            c0 = (base + r * NW) * CW
            rel = v - c0
            m = (rel >= 0) & (rel < CW)
            cnt = plsc.all_reduce_population_count(m)[0]
            plsc.store_compressed(lsts[r].at[pl.ds(offs[r], L)], rel | bidx,
                                  mask=m)
            out.append(offs[r] + cnt)
        return tuple(out)

    return lax.fori_loop(0, BATCH // L, scan, (0,) * ROUNDS)


def _extract_matches(out_hbm, buf_v, lst_v, stage_v, ssem, nmatch):
    lanes = lax.iota(jnp.int32, L)
    ngroups = (nmatch + L - 1) // L
    p0 = lst_v[pl.ds(0, L)][0]
    lst_v[pl.ds(nmatch, L)] = jnp.zeros((L,), jnp.int32) + p0

    def extract(g, _):
        packed = lst_v[pl.ds(g * L, L)]
        rel = packed & 0xFFFF
        rows = lax.shift_right_logical(packed, 16)
        for k in range(K):
            kv = jnp.zeros((L,), jnp.int32) + k
            vals = plsc.load_gather(buf_v, [kv, rel])
            plsc.store_scatter(stage_v, [lanes, kv], vals)
        pltpu.async_copy(stage_v, out_hbm.at[rows], ssem).wait()
        return _

    lax.fori_loop(0, ngroups, extract, 0)


def _extract_table(tab_hbm, tail_hbm, ids_v, out_hbm, chunk_bufs, tail_v,
                   lsts, stage_v, csem, ssem, wid, shift, tail_wid):
    base = lax.rem(wid + shift, NW)
    bufs = chunk_bufs
    sems = csem

    def _chunk_copy(r):
        c0 = pl.multiple_of((base + r * NW) * CW, CW)
        return pltpu.make_async_copy(tab_hbm.at[:, pl.ds(c0, CW)],
                                     bufs[r % 2], sems[r % 2])

    _chunk_copy(0).start()
    offs = _scan_ids3(ids_v, base, lsts)
    for r in range(ROUNDS):
        c = base + r * NW

        @pl.when(c < NCHUNK)
        def _go(r=r, c=c):
            if r + 1 < ROUNDS:
                @pl.when(c + NW < NCHUNK)
                def _pf():
                    _chunk_copy(r + 1).start()

            _chunk_copy(r).wait()
            _extract_matches(out_hbm, bufs[r % 2], lsts[r], stage_v, ssem,
                             offs[r])

    @pl.when(wid == tail_wid)
    def _tail():
        cp = pltpu.async_copy(tail_hbm, tail_v, sems[0])
        n = _scan_ids(ids_v, TAIL0, TAIL_W, lsts[0])
        cp.wait()
        _extract_matches(out_hbm, tail_v, lsts[0], stage_v, ssem, n)


def _sc_extract_body(p_hbm, q_hbm, pt_hbm, qt_hbm, u_hbm, i_hbm,
                     pu_hbm, qi_hbm,
                     u_v, i_v, chunk0_v, chunk1_v, tail_v,
                     lst0_v, lst1_v, lst2_v, lst3_v, lst4_v,
                     stage_v, csem0, csem1, ssem):
    wid = lax.axis_index("s") * NC + lax.axis_index("c")
    pltpu.sync_copy(u_hbm, u_v)
    pltpu.sync_copy(i_hbm, i_v)
    lsts = (lst0_v, lst1_v, lst2_v, lst3_v, lst4_v)
    bufs = (chunk0_v, chunk1_v)
    sems = (csem0, csem1)
    _extract_table(p_hbm, pt_hbm, u_v, pu_hbm, bufs, tail_v,
                   lsts, stage_v, sems, ssem, wid, 0, NW - 1)
    _extract_table(q_hbm, qt_hbm, i_v, qi_hbm, bufs, tail_v,
                   lsts, stage_v, sems, ssem, wid, NW // 2, NW - 2)


@functools.cache
def _sc_extract():
  return pl.kernel(
    _sc_extract_body,
    out_type=(jax.ShapeDtypeStruct((BATCH, 2 * K), jnp.float32),
              jax.ShapeDtypeStruct((BATCH, 2 * K), jnp.float32)),
    mesh=plsc.VectorSubcoreMesh(core_axis_name="c", subcore_axis_name="s",
                                num_cores=NC, num_subcores=NS),
    scratch_types=[
        pltpu.VMEM((BATCH,), jnp.int32),
        pltpu.VMEM((BATCH,), jnp.int32),
        pltpu.VMEM((K, CW), jnp.float32),
        pltpu.VMEM((K, CW), jnp.float32),
        pltpu.VMEM((K, TAIL_W), jnp.float32),
        pltpu.VMEM((LIST_N,), jnp.int32),
        pltpu.VMEM((LIST_N,), jnp.int32),
        pltpu.VMEM((LIST_N,), jnp.int32),
        pltpu.VMEM((LIST_N,), jnp.int32),
        pltpu.VMEM((LIST_N,), jnp.int32),
        pltpu.VMEM((L, 2 * K), jnp.float32),
        pltpu.SemaphoreType.DMA,
        pltpu.SemaphoreType.DMA,
        pltpu.SemaphoreType.DMA,
    ],
    compiler_params=pltpu.CompilerParams(needs_layout_passes=False),
  )


BLK_B = 1024


def _dot(a, b, dims):
    return lax.dot_general(a, b, (dims, ((), ())),
                           preferred_element_type=jnp.float32)


def _mlp_body(pu_ref, qi_ref, g_ref, w1_ref, b1_ref, w2_ref, b2_ref,
              w3_ref, b3_ref, hw_ref, qg_ref, o_ref):
    pu_t = jnp.transpose(pu_ref[...][:, :K])
    qi_item_t = jnp.transpose(qi_ref[...][:, :K])
    qi_t = qi_item_t + _dot(qg_ref[...], g_ref[...], ((1,), (1,)))
    x_t = jnp.concatenate([pu_t, qi_t], axis=0)
    z1 = _dot(w1_ref[...], x_t, ((1,), (0,))) + b1_ref[...]
    x1 = jnp.maximum(z1, 0.0)
    x2 = jnp.maximum(_dot(w2_ref[...], x1, ((1,), (0,))) + b2_ref[...], 0.0)
    x3 = jnp.maximum(_dot(w3_ref[...], x2, ((1,), (0,))) + b3_ref[...], 0.0)
    o_ref[...] = _dot(hw_ref[...], x3, ((1,), (0,)))


def _mlp_call(urows, irows, genres, W1, b1, W2, b2, W3, b3, h_w, qg):
    full = lambda shape: pl.BlockSpec(shape, lambda i: (0,) * len(shape))
    return pl.pallas_call(
        _mlp_body,
        grid=(BATCH // BLK_B,),
        in_specs=[
            pl.BlockSpec((BLK_B, 2 * K), lambda i: (i, 0)),
            pl.BlockSpec((BLK_B, 2 * K), lambda i: (i, 0)),
            pl.BlockSpec((BLK_B, N_GENRES), lambda i: (i, 0)),
            full(W1.shape), full((256, 1)),
            full(W2.shape), full((128, 1)),
            full(W3.shape), full((64, 1)),
            full(h_w.shape), full(qg.shape),
        ],
        out_specs=pl.BlockSpec((1, BLK_B), lambda i: (0, i)),
        out_shape=jax.ShapeDtypeStruct((1, BATCH), jnp.float32),
    )(urows, irows, genres, W1, b1.reshape(-1, 1), W2, b2.reshape(-1, 1),
      W3, b3.reshape(-1, 1), h_w, qg)


def kernel(user_ids, item_ids, genres_one_hot, P_w, Q_w, W1, b1, W2, b2, W3, b3, h_w):
    qg = Q_w[:, N_ITEMS:]
    urows, irows = _sc_extract()(P_w, Q_w,
                                 P_w[:, TAIL0:],
                                 Q_w[:, TAIL0:N_ITEMS],
                                 user_ids.astype(jnp.int32),
                                 item_ids.astype(jnp.int32))
    out = _mlp_call(urows, irows, genres_one_hot, W1, b1, W2, b2, W3, b3,
                    h_w, qg)
    return out.reshape(BATCH, 1)


# speedup vs baseline: 1.2030x; 1.0182x over previous
import functools

import jax
import jax.numpy as jnp
from jax import lax
from jax.experimental import pallas as pl
from jax.experimental.pallas import tpu as pltpu
from jax.experimental.pallas import tpu_sc as plsc

N_USERS = 100000
N_ITEMS = 100000
N_GENRES = 32
K = 64
BATCH = 4096

NC, NS, L = 2, 16, 16
NW = NC * NS

CW = 640
NCHUNK = N_USERS // CW
TAIL0 = NCHUNK * CW
TAIL_W = N_USERS - TAIL0
ROUNDS = -(-NCHUNK // NW)
LIST_N = BATCH + L


def _scan_ids(ids_v, c0, width, lst_v):
    lanes = lax.iota(jnp.int32, L)

    def scan(j, off):
        v = ids_v[pl.ds(j * L, L)]
        rel = v - c0
        m = (rel >= 0) & (rel < width)
        cnt = plsc.all_reduce_population_count(m)[0]
        packed = rel | ((j * L + lanes) << 16)
        plsc.store_compressed(lst_v.at[pl.ds(off, L)], packed, mask=m)
        return off + cnt

    return lax.fori_loop(0, BATCH // L, scan, 0)


def _scan_ids3(ids_v, base, lsts):
    lanes = lax.iota(jnp.int32, L)

    def scan(j, offs):
        v = ids_v[pl.ds(j * L, L)]
        bidx = (j * L + lanes) << 16
        out = []
        for r in range(ROUNDS):
            c0 = (base + r * NW) * CW
            rel = v - c0
            m = (rel >= 0) & (rel < CW)
            cnt = plsc.all_reduce_population_count(m)[0]
            plsc.store_compressed(lsts[r].at[pl.ds(offs[r], L)], rel | bidx,
                                  mask=m)
            out.append(offs[r] + cnt)
        return tuple(out)

    return lax.fori_loop(0, BATCH // L, scan, (0,) * ROUNDS)


def _extract_matches(out_hbm, buf_v, lst_v, stage_v, ssem, nmatch):
    lanes = lax.iota(jnp.int32, L)
    ngroups = (nmatch + L - 1) // L
    p0 = lst_v[pl.ds(0, L)][0]
    lst_v[pl.ds(nmatch, L)] = jnp.zeros((L,), jnp.int32) + p0

    def extract(g, _):
        packed = lst_v[pl.ds(g * L, L)]
        rel = packed & 0xFFFF
        rows = lax.shift_right_logical(packed, 16)
        for k in range(K):
            kv = jnp.zeros((L,), jnp.int32) + k
            vals = plsc.load_gather(buf_v, [kv, rel])
            plsc.store_scatter(stage_v, [lanes, kv], vals)
        pltpu.async_copy(stage_v, out_hbm.at[rows], ssem).wait()
        return _

    lax.fori_loop(0, ngroups, extract, 0)


def _extract_table(tab_hbm, tail_hbm, ids_v, out_hbm, chunk_bufs, tail_v,
                   lsts, stage_v, csem, ssem, wid, shift, tail_wid):
    base = lax.rem(wid + shift, NW)
    bufs = chunk_bufs
    sems = csem

    def _chunk_copy(r):
        c0 = pl.multiple_of((base + r * NW) * CW, CW)
        return pltpu.make_async_copy(tab_hbm.at[:, pl.ds(c0, CW)],
                                     bufs[r % 2], sems[r % 2])

    _chunk_copy(0).start()
    offs = _scan_ids3(ids_v, base, lsts)
    for r in range(ROUNDS):
        c = base + r * NW

        @pl.when(c < NCHUNK)
        def _go(r=r, c=c):
            if r + 1 < ROUNDS:
                @pl.when(c + NW < NCHUNK)
                def _pf():
                    _chunk_copy(r + 1).start()

            _chunk_copy(r).wait()
            _extract_matches(out_hbm, bufs[r % 2], lsts[r], stage_v, ssem,
                             offs[r])

    @pl.when(wid == tail_wid)
    def _tail():
        cp = pltpu.async_copy(tail_hbm, tail_v, sems[0])
        n = _scan_ids(ids_v, TAIL0, TAIL_W, lsts[0])
        cp.wait()
        _extract_matches(out_hbm, tail_v, lsts[0], stage_v, ssem, n)


def _sc_extract_body(p_hbm, q_hbm, pt_hbm, qt_hbm, u_hbm, i_hbm,
                     pu_hbm, qi_hbm,
                     u_v, i_v, chunk0_v, chunk1_v, tail_v,
                     lst0_v, lst1_v, lst2_v, lst3_v, lst4_v,
                     stage_v, csem0, csem1, ssem):
    wid = lax.axis_index("s") * NC + lax.axis_index("c")
    pltpu.sync_copy(u_hbm, u_v)
    pltpu.sync_copy(i_hbm, i_v)
    lsts = (lst0_v, lst1_v, lst2_v, lst3_v, lst4_v)
    bufs = (chunk0_v, chunk1_v)
    sems = (csem0, csem1)
    _extract_table(p_hbm, pt_hbm, u_v, pu_hbm, bufs, tail_v,
                   lsts, stage_v, sems, ssem, wid, 0, NW - 1)
    _extract_table(q_hbm, qt_hbm, i_v, qi_hbm, bufs, tail_v,
                   lsts, stage_v, sems, ssem, wid, NW // 2, NW - 2)


@functools.cache
def _sc_extract():
  return pl.kernel(
    _sc_extract_body,
    out_type=(jax.ShapeDtypeStruct((BATCH, 2 * K), jnp.float32),
              jax.ShapeDtypeStruct((BATCH, 2 * K), jnp.float32)),
    mesh=plsc.VectorSubcoreMesh(core_axis_name="c", subcore_axis_name="s",
                                num_cores=NC, num_subcores=NS),
    scratch_types=[
        pltpu.VMEM((BATCH,), jnp.int32),
        pltpu.VMEM((BATCH,), jnp.int32),
        pltpu.VMEM((K, CW), jnp.float32),
        pltpu.VMEM((K, CW), jnp.float32),
        pltpu.VMEM((K, TAIL_W), jnp.float32),
        pltpu.VMEM((LIST_N,), jnp.int32),
        pltpu.VMEM((LIST_N,), jnp.int32),
        pltpu.VMEM((LIST_N,), jnp.int32),
        pltpu.VMEM((LIST_N,), jnp.int32),
        pltpu.VMEM((LIST_N,), jnp.int32),
        pltpu.VMEM((L, 2 * K), jnp.float32),
        pltpu.SemaphoreType.DMA,
        pltpu.SemaphoreType.DMA,
        pltpu.SemaphoreType.DMA,
    ],
    compiler_params=pltpu.CompilerParams(needs_layout_passes=False),
  )


BLK_B = 2048


def _dot(a, b, dims):
    return lax.dot_general(a, b, (dims, ((), ())),
                           preferred_element_type=jnp.float32)


def _mlp_body(pu_ref, qi_ref, g_ref, w1_ref, b1_ref, w2_ref, b2_ref,
              w3_ref, b3_ref, hw_ref, qg_ref, o_ref):
    pu_t = jnp.transpose(pu_ref[...][:, :K])
    qi_item_t = jnp.transpose(qi_ref[...][:, :K])
    qi_t = qi_item_t + _dot(qg_ref[...], g_ref[...], ((1,), (1,)))
    x_t = jnp.concatenate([pu_t, qi_t], axis=0)
    z1 = _dot(w1_ref[...], x_t, ((1,), (0,))) + b1_ref[...]
    x1 = jnp.maximum(z1, 0.0)
    x2 = jnp.maximum(_dot(w2_ref[...], x1, ((1,), (0,))) + b2_ref[...], 0.0)
    x3 = jnp.maximum(_dot(w3_ref[...], x2, ((1,), (0,))) + b3_ref[...], 0.0)
    o_ref[...] = _dot(hw_ref[...], x3, ((1,), (0,)))


def _mlp_call(urows, irows, genres, W1, b1, W2, b2, W3, b3, h_w, qg):
    full = lambda shape: pl.BlockSpec(shape, lambda i: (0,) * len(shape))
    return pl.pallas_call(
        _mlp_body,
        grid=(BATCH // BLK_B,),
        in_specs=[
            pl.BlockSpec((BLK_B, 2 * K), lambda i: (i, 0)),
            pl.BlockSpec((BLK_B, 2 * K), lambda i: (i, 0)),
            pl.BlockSpec((BLK_B, N_GENRES), lambda i: (i, 0)),
            full(W1.shape), full((256, 1)),
            full(W2.shape), full((128, 1)),
            full(W3.shape), full((64, 1)),
            full(h_w.shape), full(qg.shape),
        ],
        out_specs=pl.BlockSpec((1, BLK_B), lambda i: (0, i)),
        out_shape=jax.ShapeDtypeStruct((1, BATCH), jnp.float32),
    )(urows, irows, genres, W1, b1.reshape(-1, 1), W2, b2.reshape(-1, 1),
      W3, b3.reshape(-1, 1), h_w, qg)


def kernel(user_ids, item_ids, genres_one_hot, P_w, Q_w, W1, b1, W2, b2, W3, b3, h_w):
    qg = Q_w[:, N_ITEMS:]
    urows, irows = _sc_extract()(P_w, Q_w,
                                 P_w[:, TAIL0:],
                                 Q_w[:, TAIL0:N_ITEMS],
                                 user_ids.astype(jnp.int32),
                                 item_ids.astype(jnp.int32))
    out = _mlp_call(urows, irows, genres_one_hot, W1, b1, W2, b2, W3, b3,
                    h_w, qg)
    return out.reshape(BATCH, 1)
